# Initial kernel scaffold; baseline (speedup 1.0000x reference)
import jax
import jax.numpy as jnp
from jax.experimental import pallas as pl


def kernel(x_nchw, w1, wdw, gamma, beta, run_mean, run_var):
    raise NotImplementedError("write your pallas kernel here")


# speedup vs baseline: 1.0778x; 1.0778x over previous
import functools

import jax
import jax.numpy as jnp
from jax.experimental import pallas as pl
from jax.experimental.pallas import tpu as pltpu


def _fused_body(W, x_ref, w1_ref, wdw_ref, bias_ref, o_ref):
    HW = x_ref.shape[2]
    Cout = w1_ref.shape[0]

    y = jnp.dot(w1_ref[...], x_ref[0], preferred_element_type=jnp.float32)

    col = jax.lax.broadcasted_iota(jnp.int32, (Cout, HW), 1)
    w_in_row = col & (W - 1)
    l = pltpu.roll(y, shift=1, axis=1)
    l = jnp.where(w_in_row > 0, l, 0.0)
    r = pltpu.roll(y, shift=HW - 1, axis=1)
    r = jnp.where(w_in_row < W - 1, r, 0.0)

    wdw = wdw_ref[...]

    def trow(di):
        return (l * wdw[:, 3 * di + 0:3 * di + 1]
                + y * wdw[:, 3 * di + 1:3 * di + 2]
                + r * wdw[:, 3 * di + 2:3 * di + 3])

    tm = pltpu.roll(trow(0), shift=W, axis=1)
    tp = pltpu.roll(trow(2), shift=HW - W, axis=1)
    acc = (trow(1)
           + jnp.where(col >= W, tm, 0.0)
           + jnp.where(col < HW - W, tp, 0.0))

    o_ref[0] = jnp.maximum(acc + bias_ref[...], 0.0)


def kernel(x_nchw, w1, wdw, gamma, beta, run_mean, run_var):
    eps = 1e-5
    N, Cin, H, W = x_nchw.shape
    Cout = w1.shape[0]
    HW = H * W
    f32 = jnp.float32

    inv = (gamma.astype(f32) / jnp.sqrt(run_var.astype(f32) + eps))
    bias = (beta.astype(f32) - run_mean.astype(f32) * inv)

    x2 = x_nchw.reshape(N, Cin, HW)
    w1_k = w1.astype(f32)
    wdw_k = (wdw.astype(f32) * inv[:, None, None]).reshape(Cout, 9)
    bias_k = bias[:, None]

    flops = 2 * N * HW * Cin * Cout + 19 * N * HW * Cout
    bytes_accessed = 4 * (x2.size + w1_k.size + wdw_k.size + bias_k.size
                          + N * Cout * HW)

    out = pl.pallas_call(
        functools.partial(_fused_body, W),
        out_shape=jax.ShapeDtypeStruct((N, Cout, HW), f32),
        grid=(N,),
        in_specs=[
            pl.BlockSpec((1, Cin, HW), lambda n: (n, 0, 0)),
            pl.BlockSpec((Cout, Cin), lambda n: (0, 0)),
            pl.BlockSpec((Cout, 9), lambda n: (0, 0)),
            pl.BlockSpec((Cout, 1), lambda n: (0, 0)),
        ],
        out_specs=pl.BlockSpec((1, Cout, HW), lambda n: (n, 0, 0)),
        compiler_params=pltpu.CompilerParams(
            dimension_semantics=("parallel",),
            vmem_limit_bytes=100 * 1024 * 1024,
        ),
        cost_estimate=pl.CostEstimate(
            flops=flops, transcendentals=0, bytes_accessed=bytes_accessed),
    )(x2, w1_k, wdw_k, bias_k)

    return out.reshape(N, Cout, H, W)
